# baseline (speedup 1.0000x reference)
import functools

import jax
import jax.numpy as jnp
from jax import lax
from jax.experimental import pallas as pl
from jax.experimental.pallas import tpu as pltpu
from jax.experimental.pallas import tpu_sc as plsc

N = 10000
D = 128
E = 320000

NC = 2
NS = 16
NW = NC * NS
L = 16

CHUNK = 128
CPT = 80
E_PAD = NW * CPT * CHUNK

NP = 10240
TPB = NP // NS

BLK = 1000
N_BLK = N // BLK


def _sc_mesh():
    return plsc.VectorSubcoreMesh(core_axis_name="c", subcore_axis_name="s")


_SC_PARAMS = pltpu.CompilerParams(needs_layout_passes=False)


def _deg_body(edges_hbm, out_hbm, hist, didx, tmp, acc, spart):
    c = lax.axis_index("c")
    s = lax.axis_index("s")
    wid = s * NC + c

    z16 = jnp.zeros((L,), jnp.float32)

    def zero_hist(i, _):
        hist[pl.ds(i * L, L)] = z16
        return ()
    lax.fori_loop(0, NP // L, zero_hist, ())

    pltpu.sync_copy(edges_hbm.at[1, wid], didx)

    def hist_chunk(j, _):
        for k in range(CHUNK // L):
            idx16 = didx[j, pl.ds(k * L, L)]
            cnt, last = plsc.scan_count(idx16)
            plsc.addupdate_scatter(hist, [idx16], cnt.astype(jnp.float32),
                                   mask=last)
        return ()
    lax.fori_loop(0, CPT, hist_chunk, ())

    pltpu.sync_copy(hist, spart.at[s])
    plsc.subcore_barrier()

    base = s * TPB

    def zero_acc(i, _):
        acc[pl.ds(i * L, L)] = z16
        return ()
    lax.fori_loop(0, TPB // L, zero_acc, ())

    def red(t, _):
        pltpu.sync_copy(spart.at[t, pl.ds(base, TPB)], tmp)

        def add16(k, _):
            sl = pl.ds(k * L, L)
            acc[sl] = acc[sl] + tmp[sl]
            return ()
        return lax.fori_loop(0, TPB // L, add16, ())
    lax.fori_loop(0, NS, red, ())

    pltpu.sync_copy(acc, out_hbm.at[c, pl.ds(base, TPB)])


def _deg_kernel(edges3):
    return pl.kernel(
        _deg_body,
        out_type=jax.ShapeDtypeStruct((NC, NP), jnp.float32),
        mesh=_sc_mesh(),
        compiler_params=_SC_PARAMS,
        scratch_types=[
            pltpu.VMEM((NP,), jnp.float32),
            pltpu.VMEM((CPT, CHUNK), jnp.int32),
            pltpu.VMEM((TPB,), jnp.float32),
            pltpu.VMEM((TPB,), jnp.float32),
            pltpu.VMEM_SHARED((NS, NP), jnp.float32),
        ],
    )(edges3)


def _agg_body(hs_hbm, edges_hbm, eflat_hbm, out_hbm,
              acc, rows, sidx, dst_all,
              gsem0, gsem1, isem0, isem1, ssem0, ssem1):
    c = lax.axis_index("c")
    s = lax.axis_index("s")
    wid = s * NC + c
    gsems = (gsem0, gsem1)
    isems = (isem0, isem1)
    ssems = (ssem0, ssem1)

    z16 = jnp.zeros((L,), jnp.float32)

    def zrow(i, _):
        def zcol(k, _):
            rows[0, i, pl.ds(k * L, L)] = z16
            return ()
        return lax.fori_loop(0, D // L, zcol, ())
    lax.fori_loop(0, CHUNK, zrow, ())

    def zacc(t, _):
        pltpu.sync_copy(rows.at[0],
                        acc.at[pl.ds(s * TPB + t * CHUNK, CHUNK)])
        return ()
    lax.fori_loop(0, TPB // CHUNK, zacc, ())

    pltpu.sync_copy(edges_hbm.at[1, wid], dst_all)

    plsc.subcore_barrier()

    def src_row(j):
        return eflat_hbm.at[0, wid, pl.ds(j * CHUNK, CHUNK)]

    pltpu.sync_copy(src_row(0), sidx.at[0])
    pltpu.make_async_copy(hs_hbm.at[sidx.at[0]], rows.at[0], gsems[0]).start()
    pltpu.make_async_copy(src_row(1), sidx.at[1], isems[1]).start()

    def chunk_iter(g, _):
        for b in range(2):
            j = g * 2 + b
            nb = 1 - b
            pltpu.make_async_copy(hs_hbm.at[sidx.at[b]], rows.at[b],
                                  gsems[b]).wait()

            @pl.when(j + 2 < CPT)
            def _pre_idx():
                pltpu.make_async_copy(src_row(j + 2), sidx.at[b],
                                      isems[b]).start()

            pltpu.make_async_copy(rows.at[b], acc.at[dst_all.at[j]],
                                  ssems[b]).start(add=True)

            @pl.when(j + 1 < CPT)
            def _pre_gather():
                @pl.when(j >= 1)
                def _drain_prev():
                    pltpu.make_async_copy(rows.at[nb],
                                          acc.at[dst_all.at[j - 1]],
                                          ssems[nb]).wait()

                pltpu.make_async_copy(src_row(j + 1), sidx.at[nb],
                                      isems[nb]).wait()
                pltpu.make_async_copy(hs_hbm.at[sidx.at[nb]], rows.at[nb],
                                      gsems[nb]).start()
        return ()
    lax.fori_loop(0, CPT // 2, chunk_iter, ())

    pltpu.make_async_copy(rows.at[0], acc.at[dst_all.at[CPT - 2]],
                          ssems[0]).wait()
    pltpu.make_async_copy(rows.at[1], acc.at[dst_all.at[CPT - 1]],
                          ssems[1]).wait()

    plsc.subcore_barrier()
    sl = pl.ds(s * TPB, TPB)
    pltpu.sync_copy(acc.at[sl], out_hbm.at[c, sl])


def _agg_kernel(hs, edges4, eflat):
    return pl.kernel(
        _agg_body,
        out_type=jax.ShapeDtypeStruct((NC, NP, D), jnp.float32),
        mesh=_sc_mesh(),
        compiler_params=_SC_PARAMS,
        scratch_types=[
            pltpu.VMEM_SHARED((NP, D), jnp.float32),
            pltpu.VMEM((2, CHUNK, D), jnp.float32),
            pltpu.VMEM((2, CHUNK), jnp.int32),
            pltpu.VMEM((CPT, CHUNK), jnp.int32),
            pltpu.SemaphoreType.DMA,
            pltpu.SemaphoreType.DMA,
            pltpu.SemaphoreType.DMA,
            pltpu.SemaphoreType.DMA,
            pltpu.SemaphoreType.DMA,
            pltpu.SemaphoreType.DMA,
        ],
    )(hs, edges4, eflat)


def _prep_body(e_ref, out_ref):
    out_ref[:, :E] = e_ref[...]
    npad = E_PAD - E
    it = lax.broadcasted_iota(jnp.int32, (1, npad), 1)
    out_ref[0:1, E:] = it % (2 * (NP - N))
    out_ref[1:2, E:] = N + it % (NP - N)


def _prep_kernel(ei):
    return pl.pallas_call(
        _prep_body,
        out_shape=jax.ShapeDtypeStruct((2, E_PAD), jnp.int32),
    )(ei)


def _enc_body(x_ref, w1_ref, wenc_ref, benc_ref, deg_ref, hs1_ref, xfc_ref):
    xb = x_ref[...]
    deg = deg_ref[...]
    dis = lax.rsqrt(deg[:, 0:1] + deg[:, 1:2] + 1.0)
    h1 = jnp.dot(xb, w1_ref[...], preferred_element_type=jnp.float32)
    hs1_ref[...] = h1 * dis
    xfc = jnp.dot(xb, wenc_ref[...], preferred_element_type=jnp.float32)
    xfc_ref[...] = jnp.maximum(xfc + benc_ref[...], 0.0)


def _enc_kernel(x, W1, Wenc, benc2, deg):
    return pl.pallas_call(
        _enc_body,
        grid=(N_BLK,),
        in_specs=[
            pl.BlockSpec((BLK, D), lambda i: (i, 0)),
            pl.BlockSpec((D, D), lambda i: (0, 0)),
            pl.BlockSpec((D, D), lambda i: (0, 0)),
            pl.BlockSpec((1, D), lambda i: (0, 0)),
            pl.BlockSpec((BLK, NC), lambda i: (i, 0)),
        ],
        out_specs=[
            pl.BlockSpec((BLK, D), lambda i: (i, 0)),
            pl.BlockSpec((BLK, D), lambda i: (i, 0)),
        ],
        out_shape=[
            jax.ShapeDtypeStruct((N, D), jnp.float32),
            jax.ShapeDtypeStruct((N, D), jnp.float32),
        ],
    )(x, W1, Wenc, benc2, deg)


def _mid_body(agg_ref, hs1_ref, xfc_ref, b1_ref, w2_ref, deg_ref,
              h_ref, hs2_ref):
    deg = deg_ref[...]
    dis = lax.rsqrt(deg[:, 0:1] + deg[:, 1:2] + 1.0)
    tot = agg_ref[0] + agg_ref[1] + hs1_ref[...]
    conv1 = jnp.maximum(tot * dis + b1_ref[...], 0.0)
    h = conv1 + xfc_ref[...]
    h_ref[...] = h
    g = jnp.dot(h, w2_ref[...], preferred_element_type=jnp.float32)
    hs2_ref[...] = g * dis


def _mid_kernel(agg1, hs1, xfc, b12, W2, deg):
    return pl.pallas_call(
        _mid_body,
        grid=(N_BLK,),
        in_specs=[
            pl.BlockSpec((NC, BLK, D), lambda i: (0, i, 0)),
            pl.BlockSpec((BLK, D), lambda i: (i, 0)),
            pl.BlockSpec((BLK, D), lambda i: (i, 0)),
            pl.BlockSpec((1, D), lambda i: (0, 0)),
            pl.BlockSpec((D, D), lambda i: (0, 0)),
            pl.BlockSpec((BLK, NC), lambda i: (i, 0)),
        ],
        out_specs=[
            pl.BlockSpec((BLK, D), lambda i: (i, 0)),
            pl.BlockSpec((BLK, D), lambda i: (i, 0)),
        ],
        out_shape=[
            jax.ShapeDtypeStruct((N, D), jnp.float32),
            jax.ShapeDtypeStruct((N, D), jnp.float32),
        ],
    )(agg1, hs1, xfc, b12, W2, deg)


def _head_body(agg_ref, hs2_ref, h_ref, b2_ref, deg_ref, wfc_ref, bfc_ref,
               out_ref, acc_ref):
    i = pl.program_id(0)

    @pl.when(i == 0)
    def _init():
        acc_ref[...] = jnp.zeros_like(acc_ref)

    deg = deg_ref[...]
    dis = lax.rsqrt(deg[:, 0:1] + deg[:, 1:2] + 1.0)
    tot = agg_ref[0] + agg_ref[1] + hs2_ref[...]
    conv2 = jnp.maximum(tot * dis + b2_ref[...], 0.0)
    h2 = conv2 + h_ref[...]
    acc_ref[...] = acc_ref[...] + jnp.sum(h2, axis=0, keepdims=True)

    @pl.when(i == N_BLK - 1)
    def _fin():
        pooled = acc_ref[...] / jnp.float32(N)
        logit = jnp.dot(pooled, wfc_ref[...],
                        preferred_element_type=jnp.float32) + bfc_ref[...]
        out_ref[...] = jax.nn.sigmoid(logit)


def _head_kernel(agg2, hs2, h, b22, deg, Wfc, bfc2):
    return pl.pallas_call(
        _head_body,
        grid=(N_BLK,),
        in_specs=[
            pl.BlockSpec((NC, BLK, D), lambda i: (0, i, 0)),
            pl.BlockSpec((BLK, D), lambda i: (i, 0)),
            pl.BlockSpec((BLK, D), lambda i: (i, 0)),
            pl.BlockSpec((1, D), lambda i: (0, 0)),
            pl.BlockSpec((BLK, NC), lambda i: (i, 0)),
            pl.BlockSpec((D, 1), lambda i: (0, 0)),
            pl.BlockSpec((1, 1), lambda i: (0, 0)),
        ],
        out_specs=pl.BlockSpec((1, 1), lambda i: (0, 0)),
        out_shape=jax.ShapeDtypeStruct((1, 1), jnp.float32),
        scratch_shapes=[pltpu.VMEM((1, D), jnp.float32)],
    )(agg2, hs2, h, b22, deg, Wfc, bfc2)


def kernel(x, edge_index, W1, b1, W2, b2, Wenc, benc, Wfc, bfc):
    assert x.shape == (N, D) and edge_index.shape == (2, E)

    ep = _prep_kernel(edge_index.astype(jnp.int32))
    edges4 = ep.reshape(2, NW, CPT, CHUNK)
    eflat = ep.reshape(2, NW, CPT * CHUNK)

    benc2 = benc.reshape(1, D)
    b12 = b1.reshape(1, D)
    b22 = b2.reshape(1, D)
    bfc2 = bfc.reshape(1, 1)

    deg = _deg_kernel(edges4).T
    hs1, xfc = _enc_kernel(x, W1, Wenc, benc2, deg)
    agg1 = _agg_kernel(hs1, edges4, eflat)
    h, hs2 = _mid_kernel(agg1, hs1, xfc, b12, W2, deg)
    agg2 = _agg_kernel(hs2, edges4, eflat)
    predict = _head_kernel(agg2, hs2, h, b22, deg, Wfc, bfc2)
    return predict


# speedup vs baseline: 1.0439x; 1.0439x over previous
import functools

import jax
import jax.numpy as jnp
from jax import lax
from jax.experimental import pallas as pl
from jax.experimental.pallas import tpu as pltpu
from jax.experimental.pallas import tpu_sc as plsc

N = 10000
D = 128
E = 320000

NC = 2
NS = 16
NW = NC * NS
L = 16

CHUNK = 128
CPT = 80
E_PAD = NW * CPT * CHUNK

NP = 10240
TPB = NP // NS

BLK = 2048
N_BLK = NP // BLK


def _sc_mesh():
    return plsc.VectorSubcoreMesh(core_axis_name="c", subcore_axis_name="s")


_SC_PARAMS = pltpu.CompilerParams(needs_layout_passes=False)


def _deg_body(edges_hbm, out_hbm, hist, didx, tmp, acc, spart):
    c = lax.axis_index("c")
    s = lax.axis_index("s")
    wid = s * NC + c

    z16 = jnp.zeros((L,), jnp.float32)

    def zero_hist(i, _):
        hist[pl.ds(i * L, L)] = z16
        return ()
    lax.fori_loop(0, NP // L, zero_hist, ())

    pltpu.sync_copy(edges_hbm.at[1, wid], didx)

    def hist_chunk(j, _):
        for k in range(CHUNK // L):
            idx16 = didx[j, pl.ds(k * L, L)]
            cnt, last = plsc.scan_count(idx16)
            plsc.addupdate_scatter(hist, [idx16], cnt.astype(jnp.float32),
                                   mask=last)
        return ()
    lax.fori_loop(0, CPT, hist_chunk, ())

    pltpu.sync_copy(hist, spart.at[s])
    plsc.subcore_barrier()

    base = s * TPB

    def zero_acc(i, _):
        acc[pl.ds(i * L, L)] = z16
        return ()
    lax.fori_loop(0, TPB // L, zero_acc, ())

    def red(t, _):
        pltpu.sync_copy(spart.at[t, pl.ds(base, TPB)], tmp)

        def add16(k, _):
            sl = pl.ds(k * L, L)
            acc[sl] = acc[sl] + tmp[sl]
            return ()
        return lax.fori_loop(0, TPB // L, add16, ())
    lax.fori_loop(0, NS, red, ())

    pltpu.sync_copy(acc, out_hbm.at[c, pl.ds(base, TPB)])


def _deg_kernel(edges3):
    return pl.kernel(
        _deg_body,
        out_type=jax.ShapeDtypeStruct((NC, NP), jnp.float32),
        mesh=_sc_mesh(),
        compiler_params=_SC_PARAMS,
        scratch_types=[
            pltpu.VMEM((NP,), jnp.float32),
            pltpu.VMEM((CPT, CHUNK), jnp.int32),
            pltpu.VMEM((TPB,), jnp.float32),
            pltpu.VMEM((TPB,), jnp.float32),
            pltpu.VMEM_SHARED((NS, NP), jnp.float32),
        ],
    )(edges3)


def _agg_body(hs_hbm, edges_hbm, eflat_hbm, out_hbm,
              acc, rows, sidx, dst_all,
              gsem0, gsem1, isem0, isem1, ssem0, ssem1):
    c = lax.axis_index("c")
    s = lax.axis_index("s")
    wid = s * NC + c
    gsems = (gsem0, gsem1)
    isems = (isem0, isem1)
    ssems = (ssem0, ssem1)

    z16 = jnp.zeros((L,), jnp.float32)

    def zrow(i, _):
        def zcol(k, _):
            rows[0, i, pl.ds(k * L, L)] = z16
            return ()
        return lax.fori_loop(0, D // L, zcol, ())
    lax.fori_loop(0, CHUNK, zrow, ())

    def zacc(t, _):
        pltpu.sync_copy(rows.at[0],
                        acc.at[pl.ds(s * TPB + t * CHUNK, CHUNK)])
        return ()
    lax.fori_loop(0, TPB // CHUNK, zacc, ())

    pltpu.sync_copy(edges_hbm.at[1, wid], dst_all)

    plsc.subcore_barrier()

    def src_row(j):
        return eflat_hbm.at[0, wid, pl.ds(j * CHUNK, CHUNK)]

    pltpu.sync_copy(src_row(0), sidx.at[0])
    pltpu.make_async_copy(hs_hbm.at[sidx.at[0]], rows.at[0], gsems[0]).start()
    pltpu.make_async_copy(src_row(1), sidx.at[1], isems[1]).start()

    def chunk_iter(g, _):
        for b in range(2):
            j = g * 2 + b
            nb = 1 - b
            pltpu.make_async_copy(hs_hbm.at[sidx.at[b]], rows.at[b],
                                  gsems[b]).wait()

            @pl.when(j + 2 < CPT)
            def _pre_idx():
                pltpu.make_async_copy(src_row(j + 2), sidx.at[b],
                                      isems[b]).start()

            pltpu.make_async_copy(rows.at[b], acc.at[dst_all.at[j]],
                                  ssems[b]).start(add=True)

            @pl.when(j + 1 < CPT)
            def _pre_gather():
                @pl.when(j >= 1)
                def _drain_prev():
                    pltpu.make_async_copy(rows.at[nb],
                                          acc.at[dst_all.at[j - 1]],
                                          ssems[nb]).wait()

                pltpu.make_async_copy(src_row(j + 1), sidx.at[nb],
                                      isems[nb]).wait()
                pltpu.make_async_copy(hs_hbm.at[sidx.at[nb]], rows.at[nb],
                                      gsems[nb]).start()
        return ()
    lax.fori_loop(0, CPT // 2, chunk_iter, ())

    pltpu.make_async_copy(rows.at[0], acc.at[dst_all.at[CPT - 2]],
                          ssems[0]).wait()
    pltpu.make_async_copy(rows.at[1], acc.at[dst_all.at[CPT - 1]],
                          ssems[1]).wait()

    plsc.subcore_barrier()
    sl = pl.ds(s * TPB, TPB)
    pltpu.sync_copy(acc.at[sl], out_hbm.at[c, sl])


def _agg_kernel(hs, edges4, eflat):
    return pl.kernel(
        _agg_body,
        out_type=jax.ShapeDtypeStruct((NC, NP, D), jnp.float32),
        mesh=_sc_mesh(),
        compiler_params=_SC_PARAMS,
        scratch_types=[
            pltpu.VMEM_SHARED((NP, D), jnp.float32),
            pltpu.VMEM((2, CHUNK, D), jnp.float32),
            pltpu.VMEM((2, CHUNK), jnp.int32),
            pltpu.VMEM((CPT, CHUNK), jnp.int32),
            pltpu.SemaphoreType.DMA,
            pltpu.SemaphoreType.DMA,
            pltpu.SemaphoreType.DMA,
            pltpu.SemaphoreType.DMA,
            pltpu.SemaphoreType.DMA,
            pltpu.SemaphoreType.DMA,
        ],
    )(hs, edges4, eflat)


def _prep_body(e_ref, out_ref):
    out_ref[:, :E] = e_ref[...]
    npad = E_PAD - E
    it = lax.broadcasted_iota(jnp.int32, (1, npad), 1)
    out_ref[0:1, E:] = it % (2 * (NP - N))
    out_ref[1:2, E:] = N + it % (NP - N)


def _prep_kernel(ei):
    return pl.pallas_call(
        _prep_body,
        out_shape=jax.ShapeDtypeStruct((2, E_PAD), jnp.int32),
    )(ei)


def _enc_body(x_ref, w1_ref, wenc_ref, benc_ref, deg_ref, hs1_ref, xfc_ref):
    xb = x_ref[...]
    dis = lax.rsqrt(deg_ref[0, :] + deg_ref[1, :] + 1.0)
    h1 = jnp.dot(xb, w1_ref[...], preferred_element_type=jnp.float32)
    hs1_ref[...] = h1 * dis[:, None]
    xfc = jnp.dot(xb, wenc_ref[...], preferred_element_type=jnp.float32)
    xfc_ref[...] = jnp.maximum(xfc + benc_ref[...], 0.0)


def _enc_kernel(x, W1, Wenc, benc2, deg):
    return pl.pallas_call(
        _enc_body,
        grid=(N_BLK,),
        in_specs=[
            pl.BlockSpec((BLK, D), lambda i: (i, 0)),
            pl.BlockSpec((D, D), lambda i: (0, 0)),
            pl.BlockSpec((D, D), lambda i: (0, 0)),
            pl.BlockSpec((1, D), lambda i: (0, 0)),
            pl.BlockSpec((NC, BLK), lambda i: (0, i)),
        ],
        out_specs=[
            pl.BlockSpec((BLK, D), lambda i: (i, 0)),
            pl.BlockSpec((BLK, D), lambda i: (i, 0)),
        ],
        out_shape=[
            jax.ShapeDtypeStruct((NP, D), jnp.float32),
            jax.ShapeDtypeStruct((NP, D), jnp.float32),
        ],
    )(x, W1, Wenc, benc2, deg)


def _mid_body(agg_ref, hs1_ref, xfc_ref, b1_ref, w2_ref, deg_ref,
              h_ref, hs2_ref):
    dis = lax.rsqrt(deg_ref[0, :] + deg_ref[1, :] + 1.0)
    tot = agg_ref[0] + agg_ref[1] + hs1_ref[...]
    conv1 = jnp.maximum(tot * dis[:, None] + b1_ref[...], 0.0)
    h = conv1 + xfc_ref[...]
    h_ref[...] = h
    g = jnp.dot(h, w2_ref[...], preferred_element_type=jnp.float32)
    hs2_ref[...] = g * dis[:, None]


def _mid_kernel(agg1, hs1, xfc, b12, W2, deg):
    return pl.pallas_call(
        _mid_body,
        grid=(N_BLK,),
        in_specs=[
            pl.BlockSpec((NC, BLK, D), lambda i: (0, i, 0)),
            pl.BlockSpec((BLK, D), lambda i: (i, 0)),
            pl.BlockSpec((BLK, D), lambda i: (i, 0)),
            pl.BlockSpec((1, D), lambda i: (0, 0)),
            pl.BlockSpec((D, D), lambda i: (0, 0)),
            pl.BlockSpec((NC, BLK), lambda i: (0, i)),
        ],
        out_specs=[
            pl.BlockSpec((BLK, D), lambda i: (i, 0)),
            pl.BlockSpec((BLK, D), lambda i: (i, 0)),
        ],
        out_shape=[
            jax.ShapeDtypeStruct((NP, D), jnp.float32),
            jax.ShapeDtypeStruct((NP, D), jnp.float32),
        ],
    )(agg1, hs1, xfc, b12, W2, deg)


def _head_body(agg_ref, hs2_ref, h_ref, b2_ref, deg_ref, wfc_ref, bfc_ref,
               out_ref, acc_ref):
    i = pl.program_id(0)

    @pl.when(i == 0)
    def _init():
        acc_ref[...] = jnp.zeros_like(acc_ref)

    dis = lax.rsqrt(deg_ref[0, :] + deg_ref[1, :] + 1.0)
    tot = agg_ref[0] + agg_ref[1] + hs2_ref[...]
    conv2 = jnp.maximum(tot * dis[:, None] + b2_ref[...], 0.0)
    h2 = conv2 + h_ref[...]
    rows_i = lax.broadcasted_iota(jnp.int32, (BLK, 1), 0) + i * BLK
    h2 = jnp.where(rows_i < N, h2, 0.0)
    acc_ref[...] = acc_ref[...] + jnp.sum(h2, axis=0, keepdims=True)

    @pl.when(i == N_BLK - 1)
    def _fin():
        pooled = acc_ref[...] / jnp.float32(N)
        logit = jnp.dot(pooled, wfc_ref[...],
                        preferred_element_type=jnp.float32) + bfc_ref[...]
        out_ref[...] = jax.nn.sigmoid(logit)


def _head_kernel(agg2, hs2, h, b22, deg, Wfc, bfc2):
    return pl.pallas_call(
        _head_body,
        grid=(N_BLK,),
        in_specs=[
            pl.BlockSpec((NC, BLK, D), lambda i: (0, i, 0)),
            pl.BlockSpec((BLK, D), lambda i: (i, 0)),
            pl.BlockSpec((BLK, D), lambda i: (i, 0)),
            pl.BlockSpec((1, D), lambda i: (0, 0)),
            pl.BlockSpec((NC, BLK), lambda i: (0, i)),
            pl.BlockSpec((D, 1), lambda i: (0, 0)),
            pl.BlockSpec((1, 1), lambda i: (0, 0)),
        ],
        out_specs=pl.BlockSpec((1, 1), lambda i: (0, 0)),
        out_shape=jax.ShapeDtypeStruct((1, 1), jnp.float32),
        scratch_shapes=[pltpu.VMEM((1, D), jnp.float32)],
    )(agg2, hs2, h, b22, deg, Wfc, bfc2)


def kernel(x, edge_index, W1, b1, W2, b2, Wenc, benc, Wfc, bfc):
    assert x.shape == (N, D) and edge_index.shape == (2, E)

    ep = _prep_kernel(edge_index.astype(jnp.int32))
    edges4 = ep.reshape(2, NW, CPT, CHUNK)
    eflat = ep.reshape(2, NW, CPT * CHUNK)

    benc2 = benc.reshape(1, D)
    b12 = b1.reshape(1, D)
    b22 = b2.reshape(1, D)
    bfc2 = bfc.reshape(1, 1)

    x_pad = jnp.pad(x, ((0, NP - N), (0, 0)))
    deg = _deg_kernel(edges4)
    hs1, xfc = _enc_kernel(x_pad, W1, Wenc, benc2, deg)
    agg1 = _agg_kernel(hs1, edges4, eflat)
    h, hs2 = _mid_kernel(agg1, hs1, xfc, b12, W2, deg)
    agg2 = _agg_kernel(hs2, edges4, eflat)
    predict = _head_kernel(agg2, hs2, h, b22, deg, Wfc, bfc2)
    return predict


# speedup vs baseline: 1.0661x; 1.0213x over previous
import functools

import jax
import jax.numpy as jnp
from jax import lax
from jax.experimental import pallas as pl
from jax.experimental.pallas import tpu as pltpu
from jax.experimental.pallas import tpu_sc as plsc

N = 10000
D = 128
E = 320000

NC = 2
NS = 16
NW = NC * NS
L = 16

CHUNK = 128
CPT = 80
E_PAD = NW * CPT * CHUNK

NP = 10240
TPB = NP // NS

BLK = 2048
N_BLK = NP // BLK


def _sc_mesh():
    return plsc.VectorSubcoreMesh(core_axis_name="c", subcore_axis_name="s")


_SC_PARAMS = pltpu.CompilerParams(needs_layout_passes=False)


def _deg_body(edges_hbm, out_hbm, hist, didx, tmp, acc, spart):
    c = lax.axis_index("c")
    s = lax.axis_index("s")
    wid = s * NC + c

    z16 = jnp.zeros((L,), jnp.float32)

    def zero_hist(i, _):
        hist[pl.ds(i * L, L)] = z16
        return ()
    lax.fori_loop(0, NP // L, zero_hist, ())

    pltpu.sync_copy(edges_hbm.at[1, wid], didx)

    @plsc.parallel_loop(0, CPT, unroll=2)
    def hist_chunk(j):
        for k in range(CHUNK // L):
            idx16 = didx[j, pl.ds(k * L, L)]
            cnt, last = plsc.scan_count(idx16)
            plsc.addupdate_scatter(hist, [idx16], cnt.astype(jnp.float32),
                                   mask=last)

    pltpu.sync_copy(hist, spart.at[s])
    plsc.subcore_barrier()

    base = s * TPB

    def zero_acc(i, _):
        acc[pl.ds(i * L, L)] = z16
        return ()
    lax.fori_loop(0, TPB // L, zero_acc, ())

    def red(t, _):
        pltpu.sync_copy(spart.at[t, pl.ds(base, TPB)], tmp)

        def add16(k, _):
            sl = pl.ds(k * L, L)
            acc[sl] = acc[sl] + tmp[sl]
            return ()
        return lax.fori_loop(0, TPB // L, add16, ())
    lax.fori_loop(0, NS, red, ())

    pltpu.sync_copy(acc, out_hbm.at[c, pl.ds(base, TPB)])


def _deg_kernel(edges3):
    return pl.kernel(
        _deg_body,
        out_type=jax.ShapeDtypeStruct((NC, NP), jnp.float32),
        mesh=_sc_mesh(),
        compiler_params=_SC_PARAMS,
        scratch_types=[
            pltpu.VMEM((NP,), jnp.float32),
            pltpu.VMEM((CPT, CHUNK), jnp.int32),
            pltpu.VMEM((TPB,), jnp.float32),
            pltpu.VMEM((TPB,), jnp.float32),
            pltpu.VMEM_SHARED((NS, NP), jnp.float32),
        ],
    )(edges3)


def _agg_body(hs_hbm, edges_hbm, eflat_hbm, out_hbm,
              acc, rows, sidx, dst_all,
              gsem0, gsem1, isem0, isem1, ssem0, ssem1):
    c = lax.axis_index("c")
    s = lax.axis_index("s")
    wid = s * NC + c
    gsems = (gsem0, gsem1)
    isems = (isem0, isem1)
    ssems = (ssem0, ssem1)

    z16 = jnp.zeros((L,), jnp.float32)

    def zrow(i, _):
        def zcol(k, _):
            rows[0, i, pl.ds(k * L, L)] = z16
            return ()
        return lax.fori_loop(0, D // L, zcol, ())
    lax.fori_loop(0, CHUNK, zrow, ())

    def zacc(t, _):
        pltpu.sync_copy(rows.at[0],
                        acc.at[pl.ds(s * TPB + t * CHUNK, CHUNK)])
        return ()
    lax.fori_loop(0, TPB // CHUNK, zacc, ())

    pltpu.sync_copy(edges_hbm.at[1, wid], dst_all)

    plsc.subcore_barrier()

    def src_row(j):
        return eflat_hbm.at[0, wid, pl.ds(j * CHUNK, CHUNK)]

    pltpu.sync_copy(src_row(0), sidx.at[0])
    pltpu.make_async_copy(hs_hbm.at[sidx.at[0]], rows.at[0], gsems[0]).start()
    pltpu.make_async_copy(src_row(1), sidx.at[1], isems[1]).start()

    def chunk_iter(g, _):
        for b in range(2):
            j = g * 2 + b
            nb = 1 - b
            pltpu.make_async_copy(hs_hbm.at[sidx.at[b]], rows.at[b],
                                  gsems[b]).wait()

            @pl.when(j + 2 < CPT)
            def _pre_idx():
                pltpu.make_async_copy(src_row(j + 2), sidx.at[b],
                                      isems[b]).start()

            pltpu.make_async_copy(rows.at[b], acc.at[dst_all.at[j]],
                                  ssems[b]).start(add=True)

            @pl.when(j + 1 < CPT)
            def _pre_gather():
                @pl.when(j >= 1)
                def _drain_prev():
                    pltpu.make_async_copy(rows.at[nb],
                                          acc.at[dst_all.at[j - 1]],
                                          ssems[nb]).wait()

                pltpu.make_async_copy(src_row(j + 1), sidx.at[nb],
                                      isems[nb]).wait()
                pltpu.make_async_copy(hs_hbm.at[sidx.at[nb]], rows.at[nb],
                                      gsems[nb]).start()
        return ()
    lax.fori_loop(0, CPT // 2, chunk_iter, ())

    pltpu.make_async_copy(rows.at[0], acc.at[dst_all.at[CPT - 2]],
                          ssems[0]).wait()
    pltpu.make_async_copy(rows.at[1], acc.at[dst_all.at[CPT - 1]],
                          ssems[1]).wait()

    plsc.subcore_barrier()
    sl = pl.ds(s * TPB, TPB)
    pltpu.sync_copy(acc.at[sl], out_hbm.at[c, sl])


def _agg_kernel(hs, edges4, eflat):
    return pl.kernel(
        _agg_body,
        out_type=jax.ShapeDtypeStruct((NC, NP, D), jnp.float32),
        mesh=_sc_mesh(),
        compiler_params=_SC_PARAMS,
        scratch_types=[
            pltpu.VMEM_SHARED((NP, D), jnp.float32),
            pltpu.VMEM((2, CHUNK, D), jnp.float32),
            pltpu.VMEM((2, CHUNK), jnp.int32),
            pltpu.VMEM((CPT, CHUNK), jnp.int32),
            pltpu.SemaphoreType.DMA,
            pltpu.SemaphoreType.DMA,
            pltpu.SemaphoreType.DMA,
            pltpu.SemaphoreType.DMA,
            pltpu.SemaphoreType.DMA,
            pltpu.SemaphoreType.DMA,
        ],
    )(hs, edges4, eflat)


def _prep_body(e_ref, out_ref):
    out_ref[:, :E] = e_ref[...]
    npad = E_PAD - E
    it = lax.broadcasted_iota(jnp.int32, (1, npad), 1)
    out_ref[0:1, E:] = it % (2 * (NP - N))
    out_ref[1:2, E:] = N + it % (NP - N)


def _prep_kernel(ei):
    return pl.pallas_call(
        _prep_body,
        out_shape=jax.ShapeDtypeStruct((2, E_PAD), jnp.int32),
    )(ei)


def _enc_body(x_ref, w1_ref, wenc_ref, benc_ref, deg_ref, hs1_ref, xfc_ref):
    xb = x_ref[...]
    dis = lax.rsqrt(deg_ref[0, :] + deg_ref[1, :] + 1.0)
    h1 = jnp.dot(xb, w1_ref[...], preferred_element_type=jnp.float32)
    hs1_ref[...] = h1 * dis[:, None]
    xfc = jnp.dot(xb, wenc_ref[...], preferred_element_type=jnp.float32)
    xfc_ref[...] = jnp.maximum(xfc + benc_ref[...], 0.0)


def _enc_kernel(x, W1, Wenc, benc2, deg):
    return pl.pallas_call(
        _enc_body,
        grid=(N_BLK,),
        in_specs=[
            pl.BlockSpec((BLK, D), lambda i: (i, 0)),
            pl.BlockSpec((D, D), lambda i: (0, 0)),
            pl.BlockSpec((D, D), lambda i: (0, 0)),
            pl.BlockSpec((1, D), lambda i: (0, 0)),
            pl.BlockSpec((NC, BLK), lambda i: (0, i)),
        ],
        out_specs=[
            pl.BlockSpec((BLK, D), lambda i: (i, 0)),
            pl.BlockSpec((BLK, D), lambda i: (i, 0)),
        ],
        out_shape=[
            jax.ShapeDtypeStruct((NP, D), jnp.float32),
            jax.ShapeDtypeStruct((NP, D), jnp.float32),
        ],
    )(x, W1, Wenc, benc2, deg)


def _mid_body(agg_ref, hs1_ref, xfc_ref, b1_ref, w2_ref, deg_ref,
              h_ref, hs2_ref):
    dis = lax.rsqrt(deg_ref[0, :] + deg_ref[1, :] + 1.0)
    tot = agg_ref[0] + agg_ref[1] + hs1_ref[...]
    conv1 = jnp.maximum(tot * dis[:, None] + b1_ref[...], 0.0)
    h = conv1 + xfc_ref[...]
    h_ref[...] = h
    g = jnp.dot(h, w2_ref[...], preferred_element_type=jnp.float32)
    hs2_ref[...] = g * dis[:, None]


def _mid_kernel(agg1, hs1, xfc, b12, W2, deg):
    return pl.pallas_call(
        _mid_body,
        grid=(N_BLK,),
        in_specs=[
            pl.BlockSpec((NC, BLK, D), lambda i: (0, i, 0)),
            pl.BlockSpec((BLK, D), lambda i: (i, 0)),
            pl.BlockSpec((BLK, D), lambda i: (i, 0)),
            pl.BlockSpec((1, D), lambda i: (0, 0)),
            pl.BlockSpec((D, D), lambda i: (0, 0)),
            pl.BlockSpec((NC, BLK), lambda i: (0, i)),
        ],
        out_specs=[
            pl.BlockSpec((BLK, D), lambda i: (i, 0)),
            pl.BlockSpec((BLK, D), lambda i: (i, 0)),
        ],
        out_shape=[
            jax.ShapeDtypeStruct((NP, D), jnp.float32),
            jax.ShapeDtypeStruct((NP, D), jnp.float32),
        ],
    )(agg1, hs1, xfc, b12, W2, deg)


def _head_body(agg_ref, hs2_ref, h_ref, b2_ref, deg_ref, wfc_ref, bfc_ref,
               out_ref, acc_ref):
    i = pl.program_id(0)

    @pl.when(i == 0)
    def _init():
        acc_ref[...] = jnp.zeros_like(acc_ref)

    dis = lax.rsqrt(deg_ref[0, :] + deg_ref[1, :] + 1.0)
    tot = agg_ref[0] + agg_ref[1] + hs2_ref[...]
    conv2 = jnp.maximum(tot * dis[:, None] + b2_ref[...], 0.0)
    h2 = conv2 + h_ref[...]
    rows_i = lax.broadcasted_iota(jnp.int32, (BLK, 1), 0) + i * BLK
    h2 = jnp.where(rows_i < N, h2, 0.0)
    acc_ref[...] = acc_ref[...] + jnp.sum(h2, axis=0, keepdims=True)

    @pl.when(i == N_BLK - 1)
    def _fin():
        pooled = acc_ref[...] / jnp.float32(N)
        logit = jnp.dot(pooled, wfc_ref[...],
                        preferred_element_type=jnp.float32) + bfc_ref[...]
        out_ref[...] = jax.nn.sigmoid(logit)


def _head_kernel(agg2, hs2, h, b22, deg, Wfc, bfc2):
    return pl.pallas_call(
        _head_body,
        grid=(N_BLK,),
        in_specs=[
            pl.BlockSpec((NC, BLK, D), lambda i: (0, i, 0)),
            pl.BlockSpec((BLK, D), lambda i: (i, 0)),
            pl.BlockSpec((BLK, D), lambda i: (i, 0)),
            pl.BlockSpec((1, D), lambda i: (0, 0)),
            pl.BlockSpec((NC, BLK), lambda i: (0, i)),
            pl.BlockSpec((D, 1), lambda i: (0, 0)),
            pl.BlockSpec((1, 1), lambda i: (0, 0)),
        ],
        out_specs=pl.BlockSpec((1, 1), lambda i: (0, 0)),
        out_shape=jax.ShapeDtypeStruct((1, 1), jnp.float32),
        scratch_shapes=[pltpu.VMEM((1, D), jnp.float32)],
    )(agg2, hs2, h, b22, deg, Wfc, bfc2)


def kernel(x, edge_index, W1, b1, W2, b2, Wenc, benc, Wfc, bfc):
    assert x.shape == (N, D) and edge_index.shape == (2, E)

    ep = _prep_kernel(edge_index.astype(jnp.int32))
    edges4 = ep.reshape(2, NW, CPT, CHUNK)
    eflat = ep.reshape(2, NW, CPT * CHUNK)

    benc2 = benc.reshape(1, D)
    b12 = b1.reshape(1, D)
    b22 = b2.reshape(1, D)
    bfc2 = bfc.reshape(1, 1)

    x_pad = jnp.pad(x, ((0, NP - N), (0, 0)))
    deg = _deg_kernel(edges4)
    hs1, xfc = _enc_kernel(x_pad, W1, Wenc, benc2, deg)
    agg1 = _agg_kernel(hs1, edges4, eflat)
    h, hs2 = _mid_kernel(agg1, hs1, xfc, b12, W2, deg)
    agg2 = _agg_kernel(hs2, edges4, eflat)
    predict = _head_kernel(agg2, hs2, h, b22, deg, Wfc, bfc2)
    return predict


# speedup vs baseline: 1.0758x; 1.0091x over previous
import functools

import jax
import jax.numpy as jnp
from jax import lax
from jax.experimental import pallas as pl
from jax.experimental.pallas import tpu as pltpu
from jax.experimental.pallas import tpu_sc as plsc

N = 10000
D = 128
E = 320000

NC = 2
NS = 16
NW = NC * NS
L = 16

CHUNK = 128
CPT = 80
E_PAD = NW * CPT * CHUNK

NP = 10240
TPB = NP // NS

BLK = 2048
N_BLK = NP // BLK


def _sc_mesh():
    return plsc.VectorSubcoreMesh(core_axis_name="c", subcore_axis_name="s")


_SC_PARAMS = pltpu.CompilerParams(needs_layout_passes=False)


def _deg_body(edges_hbm, out_hbm, hist, didx, tmp, acc, spart):
    c = lax.axis_index("c")
    s = lax.axis_index("s")
    wid = s * NC + c

    z16 = jnp.zeros((L,), jnp.float32)

    def zero_hist(i, _):
        hist[pl.ds(i * L, L)] = z16
        return ()
    lax.fori_loop(0, NP // L, zero_hist, ())

    pltpu.sync_copy(edges_hbm.at[1, wid], didx)

    @plsc.parallel_loop(0, CPT, unroll=4)
    def hist_chunk(j):
        for k in range(CHUNK // L):
            idx16 = didx[j, pl.ds(k * L, L)]
            cnt, last = plsc.scan_count(idx16)
            plsc.addupdate_scatter(hist, [idx16], cnt.astype(jnp.float32),
                                   mask=last)

    pltpu.sync_copy(hist, spart.at[s])
    plsc.subcore_barrier()

    base = s * TPB

    def zero_acc(i, _):
        acc[pl.ds(i * L, L)] = z16
        return ()
    lax.fori_loop(0, TPB // L, zero_acc, ())

    def red(t, _):
        pltpu.sync_copy(spart.at[t, pl.ds(base, TPB)], tmp)

        @plsc.parallel_loop(0, TPB // L, unroll=4)
        def add16(k):
            sl = pl.ds(k * L, L)
            acc[sl] = acc[sl] + tmp[sl]
        return ()
    lax.fori_loop(0, NS, red, ())

    pltpu.sync_copy(acc, out_hbm.at[c, pl.ds(base, TPB)])


def _deg_kernel(edges3):
    return pl.kernel(
        _deg_body,
        out_type=jax.ShapeDtypeStruct((NC, NP), jnp.float32),
        mesh=_sc_mesh(),
        compiler_params=_SC_PARAMS,
        scratch_types=[
            pltpu.VMEM((NP,), jnp.float32),
            pltpu.VMEM((CPT, CHUNK), jnp.int32),
            pltpu.VMEM((TPB,), jnp.float32),
            pltpu.VMEM((TPB,), jnp.float32),
            pltpu.VMEM_SHARED((NS, NP), jnp.float32),
        ],
    )(edges3)


def _agg_body(hs_hbm, edges_hbm, eflat_hbm, out_hbm,
              acc, rows, sidx, dst_all,
              gsem0, gsem1, isem0, isem1, ssem0, ssem1):
    c = lax.axis_index("c")
    s = lax.axis_index("s")
    wid = s * NC + c
    gsems = (gsem0, gsem1)
    isems = (isem0, isem1)
    ssems = (ssem0, ssem1)

    z16 = jnp.zeros((L,), jnp.float32)

    def zrow(i, _):
        def zcol(k, _):
            rows[0, i, pl.ds(k * L, L)] = z16
            return ()
        return lax.fori_loop(0, D // L, zcol, ())
    lax.fori_loop(0, CHUNK, zrow, ())

    def zacc(t, _):
        pltpu.sync_copy(rows.at[0],
                        acc.at[pl.ds(s * TPB + t * CHUNK, CHUNK)])
        return ()
    lax.fori_loop(0, TPB // CHUNK, zacc, ())

    pltpu.sync_copy(edges_hbm.at[1, wid], dst_all)

    plsc.subcore_barrier()

    def src_row(j):
        return eflat_hbm.at[0, wid, pl.ds(j * CHUNK, CHUNK)]

    pltpu.sync_copy(src_row(0), sidx.at[0])
    pltpu.make_async_copy(hs_hbm.at[sidx.at[0]], rows.at[0], gsems[0]).start()
    pltpu.make_async_copy(src_row(1), sidx.at[1], isems[1]).start()

    def chunk_iter(g, _):
        for b in range(2):
            j = g * 2 + b
            nb = 1 - b
            pltpu.make_async_copy(hs_hbm.at[sidx.at[b]], rows.at[b],
                                  gsems[b]).wait()

            @pl.when(j + 2 < CPT)
            def _pre_idx():
                pltpu.make_async_copy(src_row(j + 2), sidx.at[b],
                                      isems[b]).start()

            pltpu.make_async_copy(rows.at[b], acc.at[dst_all.at[j]],
                                  ssems[b]).start(add=True)

            @pl.when(j + 1 < CPT)
            def _pre_gather():
                @pl.when(j >= 1)
                def _drain_prev():
                    pltpu.make_async_copy(rows.at[nb],
                                          acc.at[dst_all.at[j - 1]],
                                          ssems[nb]).wait()

                pltpu.make_async_copy(src_row(j + 1), sidx.at[nb],
                                      isems[nb]).wait()
                pltpu.make_async_copy(hs_hbm.at[sidx.at[nb]], rows.at[nb],
                                      gsems[nb]).start()
        return ()
    lax.fori_loop(0, CPT // 2, chunk_iter, ())

    pltpu.make_async_copy(rows.at[0], acc.at[dst_all.at[CPT - 2]],
                          ssems[0]).wait()
    pltpu.make_async_copy(rows.at[1], acc.at[dst_all.at[CPT - 1]],
                          ssems[1]).wait()

    plsc.subcore_barrier()
    sl = pl.ds(s * TPB, TPB)
    pltpu.sync_copy(acc.at[sl], out_hbm.at[c, sl])


def _agg_kernel(hs, edges4, eflat):
    return pl.kernel(
        _agg_body,
        out_type=jax.ShapeDtypeStruct((NC, NP, D), jnp.float32),
        mesh=_sc_mesh(),
        compiler_params=_SC_PARAMS,
        scratch_types=[
            pltpu.VMEM_SHARED((NP, D), jnp.float32),
            pltpu.VMEM((2, CHUNK, D), jnp.float32),
            pltpu.VMEM((2, CHUNK), jnp.int32),
            pltpu.VMEM((CPT, CHUNK), jnp.int32),
            pltpu.SemaphoreType.DMA,
            pltpu.SemaphoreType.DMA,
            pltpu.SemaphoreType.DMA,
            pltpu.SemaphoreType.DMA,
            pltpu.SemaphoreType.DMA,
            pltpu.SemaphoreType.DMA,
        ],
    )(hs, edges4, eflat)


def _prep_body(e_ref, out_ref):
    out_ref[:, :E] = e_ref[...]
    npad = E_PAD - E
    it = lax.broadcasted_iota(jnp.int32, (1, npad), 1)
    out_ref[0:1, E:] = it % (2 * (NP - N))
    out_ref[1:2, E:] = N + it % (NP - N)


def _prep_kernel(ei):
    return pl.pallas_call(
        _prep_body,
        out_shape=jax.ShapeDtypeStruct((2, E_PAD), jnp.int32),
    )(ei)


def _enc_body(x_ref, w1_ref, wenc_ref, benc_ref, deg_ref, hs1_ref, xfc_ref):
    xb = x_ref[...]
    dis = lax.rsqrt(deg_ref[0, :] + deg_ref[1, :] + 1.0)
    h1 = jnp.dot(xb, w1_ref[...], preferred_element_type=jnp.float32)
    hs1_ref[...] = h1 * dis[:, None]
    xfc = jnp.dot(xb, wenc_ref[...], preferred_element_type=jnp.float32)
    xfc_ref[...] = jnp.maximum(xfc + benc_ref[...], 0.0)


def _enc_kernel(x, W1, Wenc, benc2, deg):
    return pl.pallas_call(
        _enc_body,
        grid=(N_BLK,),
        in_specs=[
            pl.BlockSpec((BLK, D), lambda i: (i, 0)),
            pl.BlockSpec((D, D), lambda i: (0, 0)),
            pl.BlockSpec((D, D), lambda i: (0, 0)),
            pl.BlockSpec((1, D), lambda i: (0, 0)),
            pl.BlockSpec((NC, BLK), lambda i: (0, i)),
        ],
        out_specs=[
            pl.BlockSpec((BLK, D), lambda i: (i, 0)),
            pl.BlockSpec((BLK, D), lambda i: (i, 0)),
        ],
        out_shape=[
            jax.ShapeDtypeStruct((NP, D), jnp.float32),
            jax.ShapeDtypeStruct((NP, D), jnp.float32),
        ],
    )(x, W1, Wenc, benc2, deg)


def _mid_body(agg_ref, hs1_ref, xfc_ref, b1_ref, w2_ref, deg_ref,
              h_ref, hs2_ref):
    dis = lax.rsqrt(deg_ref[0, :] + deg_ref[1, :] + 1.0)
    tot = agg_ref[0] + agg_ref[1] + hs1_ref[...]
    conv1 = jnp.maximum(tot * dis[:, None] + b1_ref[...], 0.0)
    h = conv1 + xfc_ref[...]
    h_ref[...] = h
    g = jnp.dot(h, w2_ref[...], preferred_element_type=jnp.float32)
    hs2_ref[...] = g * dis[:, None]


def _mid_kernel(agg1, hs1, xfc, b12, W2, deg):
    return pl.pallas_call(
        _mid_body,
        grid=(N_BLK,),
        in_specs=[
            pl.BlockSpec((NC, BLK, D), lambda i: (0, i, 0)),
            pl.BlockSpec((BLK, D), lambda i: (i, 0)),
            pl.BlockSpec((BLK, D), lambda i: (i, 0)),
            pl.BlockSpec((1, D), lambda i: (0, 0)),
            pl.BlockSpec((D, D), lambda i: (0, 0)),
            pl.BlockSpec((NC, BLK), lambda i: (0, i)),
        ],
        out_specs=[
            pl.BlockSpec((BLK, D), lambda i: (i, 0)),
            pl.BlockSpec((BLK, D), lambda i: (i, 0)),
        ],
        out_shape=[
            jax.ShapeDtypeStruct((NP, D), jnp.float32),
            jax.ShapeDtypeStruct((NP, D), jnp.float32),
        ],
    )(agg1, hs1, xfc, b12, W2, deg)


def _head_body(agg_ref, hs2_ref, h_ref, b2_ref, deg_ref, wfc_ref, bfc_ref,
               out_ref, acc_ref):
    i = pl.program_id(0)

    @pl.when(i == 0)
    def _init():
        acc_ref[...] = jnp.zeros_like(acc_ref)

    dis = lax.rsqrt(deg_ref[0, :] + deg_ref[1, :] + 1.0)
    tot = agg_ref[0] + agg_ref[1] + hs2_ref[...]
    conv2 = jnp.maximum(tot * dis[:, None] + b2_ref[...], 0.0)
    h2 = conv2 + h_ref[...]
    rows_i = lax.broadcasted_iota(jnp.int32, (BLK, 1), 0) + i * BLK
    h2 = jnp.where(rows_i < N, h2, 0.0)
    acc_ref[...] = acc_ref[...] + jnp.sum(h2, axis=0, keepdims=True)

    @pl.when(i == N_BLK - 1)
    def _fin():
        pooled = acc_ref[...] / jnp.float32(N)
        logit = jnp.dot(pooled, wfc_ref[...],
                        preferred_element_type=jnp.float32) + bfc_ref[...]
        out_ref[...] = jax.nn.sigmoid(logit)


def _head_kernel(agg2, hs2, h, b22, deg, Wfc, bfc2):
    return pl.pallas_call(
        _head_body,
        grid=(N_BLK,),
        in_specs=[
            pl.BlockSpec((NC, BLK, D), lambda i: (0, i, 0)),
            pl.BlockSpec((BLK, D), lambda i: (i, 0)),
            pl.BlockSpec((BLK, D), lambda i: (i, 0)),
            pl.BlockSpec((1, D), lambda i: (0, 0)),
            pl.BlockSpec((NC, BLK), lambda i: (0, i)),
            pl.BlockSpec((D, 1), lambda i: (0, 0)),
            pl.BlockSpec((1, 1), lambda i: (0, 0)),
        ],
        out_specs=pl.BlockSpec((1, 1), lambda i: (0, 0)),
        out_shape=jax.ShapeDtypeStruct((1, 1), jnp.float32),
        scratch_shapes=[pltpu.VMEM((1, D), jnp.float32)],
    )(agg2, hs2, h, b22, deg, Wfc, bfc2)


def kernel(x, edge_index, W1, b1, W2, b2, Wenc, benc, Wfc, bfc):
    assert x.shape == (N, D) and edge_index.shape == (2, E)

    ep = _prep_kernel(edge_index.astype(jnp.int32))
    edges4 = ep.reshape(2, NW, CPT, CHUNK)
    eflat = ep.reshape(2, NW, CPT * CHUNK)

    benc2 = benc.reshape(1, D)
    b12 = b1.reshape(1, D)
    b22 = b2.reshape(1, D)
    bfc2 = bfc.reshape(1, 1)

    x_pad = jnp.pad(x, ((0, NP - N), (0, 0)))
    deg = _deg_kernel(edges4)
    hs1, xfc = _enc_kernel(x_pad, W1, Wenc, benc2, deg)
    agg1 = _agg_kernel(hs1, edges4, eflat)
    h, hs2 = _mid_kernel(agg1, hs1, xfc, b12, W2, deg)
    agg2 = _agg_kernel(hs2, edges4, eflat)
    predict = _head_kernel(agg2, hs2, h, b22, deg, Wfc, bfc2)
    return predict
